# Initial kernel scaffold; baseline (speedup 1.0000x reference)
import jax
import jax.numpy as jnp
from jax.experimental import pallas as pl


def kernel(features, nodes, neigh_idx, weight):
    raise NotImplementedError("write your pallas kernel here")


# speedup vs baseline: 1.0544x; 1.0544x over previous
import functools

import jax
import jax.numpy as jnp
from jax import lax
from jax.experimental import pallas as pl
from jax.experimental.pallas import tpu as pltpu
from jax.experimental.pallas import tpu_sc as plsc

B = 1024
D = 128
E = 128
S = 16
NC = 2
NS = 16
NW = NC * NS
BPW = B // NW
LANES = 16


def _sc_gather_mean(features, nodes, neigh_flat):
    mesh = plsc.VectorSubcoreMesh(core_axis_name="c", subcore_axis_name="s")

    @functools.partial(
        pl.kernel,
        mesh=mesh,
        out_type=[
            jax.ShapeDtypeStruct((B, D), jnp.float32),
            jax.ShapeDtypeStruct((B, D), jnp.float32),
        ],
        scratch_types=[
            pltpu.VMEM((BPW,), jnp.int32),
            pltpu.VMEM((BPW * S,), jnp.int32),
            pltpu.VMEM((BPW, D), jnp.float32),
            pltpu.VMEM((BPW * S, D), jnp.float32),
            pltpu.VMEM((BPW, D), jnp.float32),
            pltpu.SemaphoreType.DMA,
        ],
    )
    def k(feat_hbm, nodes_hbm, neigh_hbm, self_out, mean_out,
          sidx, nidx, srows, nrows, mbuf, sem):
        wid = lax.axis_index("s") * NC + lax.axis_index("c")
        base = wid * BPW

        pltpu.sync_copy(nodes_hbm.at[pl.ds(base, BPW)], sidx)
        pltpu.sync_copy(neigh_hbm.at[pl.ds(base * S, BPW * S)], nidx)

        copies = [pltpu.async_copy(feat_hbm.at[sidx], srows, sem)]
        n_chunks = (BPW * S) // 128
        for j in range(n_chunks):
            copies.append(
                pltpu.async_copy(
                    feat_hbm.at[nidx.at[pl.ds(j * 128, 128)]],
                    nrows.at[pl.ds(j * 128, 128)],
                    sem,
                )
            )
        for c in copies:
            c.wait()

        inv_s = jnp.float32(1.0 / S)

        def body(j, carry):
            row0 = j * S
            for c in range(D // LANES):
                sl = pl.ds(c * LANES, LANES)
                acc = nrows[row0, sl]
                for s in range(1, S):
                    acc = acc + nrows[row0 + s, sl]
                mbuf[j, sl] = acc * inv_s
            return carry

        lax.fori_loop(0, BPW, body, 0)

        pltpu.sync_copy(srows, self_out.at[pl.ds(base, BPW)])
        pltpu.sync_copy(mbuf, mean_out.at[pl.ds(base, BPW)])

    return k(features, nodes, neigh_flat)


def _tc_encode(self_f, mean_f, weight):
    BB = 16
    grid = B // BB
    K = BB * 2 * D

    def body(s_ref, m_ref, w_ref, o_ref):
        comb = jnp.concatenate([s_ref[...], m_ref[...]], axis=1)
        w = w_ref[...].reshape(K, E)
        tiled = jnp.tile(comb, (1, BB))
        row = lax.broadcasted_iota(jnp.int32, (BB, K), 0)
        grp = lax.broadcasted_iota(jnp.int32, (BB, K), 1) // (2 * D)
        cdiag = jnp.where(row == grp, tiled, jnp.float32(0.0))
        out = lax.dot_general(cdiag, w, (((1,), (0,)), ((), ())),
                              preferred_element_type=jnp.float32)
        o_ref[...] = jnp.maximum(out, jnp.float32(0.0))

    return pl.pallas_call(
        body,
        grid=(grid,),
        in_specs=[
            pl.BlockSpec((BB, D), lambda i: (i, 0)),
            pl.BlockSpec((BB, D), lambda i: (i, 0)),
            pl.BlockSpec((BB, 2 * D, E), lambda i: (i, 0, 0)),
        ],
        out_specs=pl.BlockSpec((BB, E), lambda i: (i, 0)),
        out_shape=jax.ShapeDtypeStruct((B, E), jnp.float32),
    )(self_f, mean_f, weight)


def kernel(features, nodes, neigh_idx, weight):
    nodes = nodes.astype(jnp.int32)
    neigh_flat = neigh_idx.astype(jnp.int32).reshape(-1)
    self_f, mean_f = _sc_gather_mean(features, nodes, neigh_flat)
    return _tc_encode(self_f, mean_f, weight)
